# Initial kernel scaffold; baseline (speedup 1.0000x reference)
import jax
import jax.numpy as jnp
from jax import lax
from jax.experimental import pallas as pl
from jax.experimental.pallas import tpu as pltpu
from jax.experimental.pallas import tpu_sc as plsc

B, C, W = 32, 16, 65536
N = 32768
L = 16
CH = 16384
NCHUNK = N // CH
ITERS = CH // L


def _tec_body(data_hbm, param_hbm, out_hbm, tex, pbuf, obuf):
    nc = 2
    b = lax.axis_index("s") * nc + lax.axis_index("c")
    pltpu.sync_copy(param_hbm.at[b], pbuf)
    for c in range(C):
        pltpu.sync_copy(data_hbm.at[b, c], tex)
        for h in range(NCHUNK):
            def body(j, carry):
                p = pbuf[pl.ds(h * CH + j * L, L)]
                x = p * float(W - 1)
                i0 = x.astype(jnp.int32)
                w = x - i0.astype(jnp.float32)
                i1 = jnp.minimum(i0 + 1, W - 1)
                g0 = plsc.load_gather(tex, [i0])
                g1 = plsc.load_gather(tex, [i1])
                obuf[pl.ds(j * L, L)] = g0 * (1.0 - w) + g1 * w
                return carry

            lax.fori_loop(0, ITERS, body, 0)
            pltpu.sync_copy(obuf, out_hbm.at[b, c, pl.ds(h * CH, CH)])


def kernel(data, param):
    mesh = plsc.VectorSubcoreMesh(core_axis_name="c", subcore_axis_name="s")
    f = pl.kernel(
        _tec_body,
        out_type=jax.ShapeDtypeStruct((B, C, N), jnp.float32),
        mesh=mesh,
        scratch_types=[
            pltpu.VMEM((W,), jnp.float32),
            pltpu.VMEM((N,), jnp.float32),
            pltpu.VMEM((CH,), jnp.float32),
        ],
    )
    return f(data, param)


# speedup vs baseline: 3.2768x; 3.2768x over previous
import jax
import jax.numpy as jnp
from jax import lax
from jax.experimental import pallas as pl
from jax.experimental.pallas import tpu as pltpu
from jax.experimental.pallas import tpu_sc as plsc

B, C, W = 32, 16, 65536
N = 32768
L = 16
CH = 16384
NCHUNK = N // CH
ITERS = CH // L


def _tec_body(data_hbm, param_hbm, out_hbm, tex, pbuf, obuf):
    nc = 2
    b = lax.axis_index("s") * nc + lax.axis_index("c")
    pltpu.sync_copy(param_hbm.at[b], pbuf)
    for c in range(C):
        pltpu.sync_copy(data_hbm.at[b, c], tex)
        for h in range(NCHUNK):
            def body(j, carry):
                p = pbuf[pl.ds(h * CH + j * L, L)]
                x = p * float(W - 1)
                i0 = x.astype(jnp.int32)
                w = x - i0.astype(jnp.float32)
                i1 = jnp.minimum(i0 + 1, W - 1)
                g0 = plsc.load_gather(tex, [i0])
                g1 = plsc.load_gather(tex, [i1])
                obuf[pl.ds(j * L, L)] = g0 * (1.0 - w) + g1 * w
                return carry

            lax.fori_loop(0, ITERS, body, 0)
            pltpu.sync_copy(obuf, out_hbm.at[b, c, pl.ds(h * CH, CH)])


def kernel(data, param):
    mesh = plsc.VectorSubcoreMesh(core_axis_name="c", subcore_axis_name="s")
    f = pl.kernel(
        _tec_body,
        out_type=jax.ShapeDtypeStruct((B, C, N), jnp.float32),
        mesh=mesh,
        compiler_params=pltpu.CompilerParams(needs_layout_passes=False),
        scratch_types=[
            pltpu.VMEM((W,), jnp.float32),
            pltpu.VMEM((N,), jnp.float32),
            pltpu.VMEM((CH,), jnp.float32),
        ],
    )
    return f(data, param)


# speedup vs baseline: 4.7818x; 1.4593x over previous
import jax
import jax.numpy as jnp
from jax import lax
from jax.experimental import pallas as pl
from jax.experimental.pallas import tpu as pltpu
from jax.experimental.pallas import tpu_sc as plsc

B, C, W = 32, 16, 65536
N = 32768
L = 16
CH = 16384
NCHUNK = N // CH
ITERS = CH // L


U = 8


def _tec_body(data_hbm, param_hbm, out_hbm, tex, pbuf, obuf):
    nc = 2
    b = lax.axis_index("s") * nc + lax.axis_index("c")
    pltpu.sync_copy(param_hbm.at[b], pbuf)

    def chan(c, carry):
        pltpu.sync_copy(data_hbm.at[b, c], tex)
        for h in range(NCHUNK):
            def body(j, carry2):
                base = h * CH + j * (U * L)
                for u in range(U):
                    p = pbuf[pl.ds(base + u * L, L)]
                    x = p * float(W - 1)
                    i0 = x.astype(jnp.int32)
                    w = x - i0.astype(jnp.float32)
                    i1 = jnp.minimum(i0 + 1, W - 1)
                    g0 = plsc.load_gather(tex, [i0])
                    g1 = plsc.load_gather(tex, [i1])
                    obuf[pl.ds(j * (U * L) + u * L, L)] = (
                        g0 * (1.0 - w) + g1 * w)
                return carry2

            lax.fori_loop(0, ITERS // U, body, 0)
            pltpu.sync_copy(obuf, out_hbm.at[b, c, pl.ds(h * CH, CH)])
        return carry

    lax.fori_loop(0, C, chan, 0)


def kernel(data, param):
    mesh = plsc.VectorSubcoreMesh(core_axis_name="c", subcore_axis_name="s")
    f = pl.kernel(
        _tec_body,
        out_type=jax.ShapeDtypeStruct((B, C, N), jnp.float32),
        mesh=mesh,
        compiler_params=pltpu.CompilerParams(needs_layout_passes=False),
        scratch_types=[
            pltpu.VMEM((W,), jnp.float32),
            pltpu.VMEM((N,), jnp.float32),
            pltpu.VMEM((CH,), jnp.float32),
        ],
    )
    return f(data, param)
